# baseline (speedup 1.0000x reference)
import functools

import jax
import jax.numpy as jnp
from jax import lax
from jax.experimental import pallas as pl
from jax.experimental.pallas import tpu as pltpu
from jax.experimental.pallas import tpu_sc as plsc

N = 100000
E = 6400000
NF = 8
H = 64
VOCAB = 5
NG = 1024
GHI = 32
GLO = 32

NC = 2
NS = 16
NW = NC * NS
SEG = 128
K = 8
CHUNK = K * SEG
NCHUNK = E // CHUNK
NSEGTOT = E // SEG
NPAD = 100096
ROWS_PER_TILE = NPAD // NS

BN = 2000
NBLK = N // BN


def _build_x_body(pos_ref, z_ref, emb_ref, x_ref):
    z = z_ref[0, 0, :]
    onehot = (z[:, None] == lax.broadcasted_iota(jnp.int32, (BN, VOCAB), 1))
    xe = jnp.dot(onehot.astype(jnp.float32), emb_ref[...],
                 preferred_element_type=jnp.float32)
    x_ref[...] = jnp.concatenate([pos_ref[...], xe], axis=1)


def _build_x(pos, z3, emb):
    return pl.pallas_call(
        _build_x_body,
        grid=(NBLK,),
        in_specs=[
            pl.BlockSpec((BN, 3), lambda i: (i, 0)),
            pl.BlockSpec((1, 1, BN), lambda i: (i, 0, 0)),
            pl.BlockSpec((VOCAB, VOCAB), lambda i: (0, 0)),
        ],
        out_specs=pl.BlockSpec((BN, NF), lambda i: (i, 0)),
        out_shape=jax.ShapeDtypeStruct((N, NF), jnp.float32),
    )(pos, z3, emb)


def _edge_agg_body(x_hbm, ei_hbm, zeros_hbm, out_hbm,
                   idx_v, rows_v, agg_sh, sem_i, sem_g, sem_s):
    c = lax.axis_index("c")
    s = lax.axis_index("s")
    wid = s * NC + c

    r0 = s * ROWS_PER_TILE
    pltpu.sync_copy(zeros_hbm.at[pl.ds(r0, ROWS_PER_TILE)],
                    agg_sh.at[pl.ds(r0, ROWS_PER_TILE)])
    plsc.subcore_barrier()

    base = NCHUNK // NW
    extra = NCHUNK - base * NW
    nloc = base + jnp.where(wid < extra, 1, 0)

    def idx_copy(i, slot, half):
        return pltpu.make_async_copy(
            ei_hbm.at[half, pl.ds((i * NW + wid) * CHUNK, CHUNK)],
            idx_v.at[slot, half], sem_i)

    def scat_copy(j, slot, b):
        return pltpu.make_async_copy(
            rows_v.at[b, j], agg_sh.at[idx_v.at[slot, 1, pl.ds(j * SEG, SEG)]],
            sem_s)

    idx_copy(0, 0, 0).start()
    idx_copy(0, 0, 1).start()

    def body(i, carry):
        b = jnp.bitwise_and(i, 1)
        s3 = lax.rem(i, 3)

        @pl.when(i + 1 < nloc)
        def _():
            n3 = lax.rem(i + 1, 3)
            idx_copy(i + 1, n3, 0).start()
            idx_copy(i + 1, n3, 1).start()

        idx_copy(i, s3, 0).wait()
        idx_copy(i, s3, 1).wait()
        gs = [pltpu.async_copy(
                  x_hbm.at[idx_v.at[s3, 0, pl.ds(j * SEG, SEG)]],
                  rows_v.at[b, j], sem_g) for j in range(K)]

        @pl.when(i > 0)
        def _():
            pb = jnp.bitwise_xor(b, 1)
            p3 = lax.rem(i + 2, 3)
            for j in range(K):
                scat_copy(j, p3, pb).wait()

        for cp in gs:
            cp.wait()
        for j in range(K):
            scat_copy(j, s3, b).start(add=True)
        return carry

    lax.fori_loop(0, nloc, body, 0)
    lb = jnp.bitwise_and(nloc - 1, 1)
    l3 = lax.rem(nloc - 1, 3)
    for j in range(K):
        scat_copy(j, l3, lb).wait()

    plsc.subcore_barrier()
    pltpu.sync_copy(agg_sh.at[pl.ds(r0, ROWS_PER_TILE)],
                    out_hbm.at[c, pl.ds(r0, ROWS_PER_TILE)])


def _edge_agg(x, ei3, zeros):
    mesh = plsc.VectorSubcoreMesh(core_axis_name="c", subcore_axis_name="s")
    fn = functools.partial(
        pl.kernel,
        out_type=jax.ShapeDtypeStruct((NC, NPAD, NF), jnp.float32),
        mesh=mesh,
        scratch_types=[
            pltpu.VMEM((3, 2, CHUNK), jnp.int32),
            pltpu.VMEM((2, K, SEG, NF), jnp.float32),
            pltpu.VMEM_SHARED((NPAD, NF), jnp.float32),
            pltpu.SemaphoreType.DMA,
            pltpu.SemaphoreType.DMA,
            pltpu.SemaphoreType.DMA,
        ],
        compiler_params=pltpu.CompilerParams(use_tc_tiling_on_sc=False),
    )(_edge_agg_body)
    return fn(x, ei3, zeros)


def _mlp_pool_body(x_ref, p_ref, b3_ref, W1_ref, b1_ref, W2_ref, b2_ref,
                   Wfc_ref, bfc_ref, out_ref):
    i = pl.program_id(0)
    h = x_ref[...] + p_ref[0] + p_ref[1]
    a1 = jnp.maximum(
        jnp.dot(h, W1_ref[...], preferred_element_type=jnp.float32)
        + b1_ref[...], 0.0)
    a2 = jnp.maximum(
        jnp.dot(a1, W2_ref[...], preferred_element_type=jnp.float32)
        + b2_ref[...], 0.0)
    f = jnp.dot(a2, Wfc_ref[...], preferred_element_type=jnp.float32)

    seg = b3_ref[0, 0, :]
    lo = jnp.bitwise_and(seg, GLO - 1)
    hi = jnp.right_shift(seg, 5)
    oh_lo = (lo[:, None] == lax.broadcasted_iota(jnp.int32, (BN, GLO), 1))
    oh_hi_t = (hi[None, :] == lax.broadcasted_iota(jnp.int32, (GHI, BN), 0))
    contrib = jnp.dot(oh_hi_t.astype(jnp.float32),
                      f * oh_lo.astype(jnp.float32),
                      preferred_element_type=jnp.float32)

    @pl.when(i == 0)
    def _():
        out_ref[...] = contrib + bfc_ref[...]

    @pl.when(i != 0)
    def _():
        out_ref[...] += contrib


def _mlp_pool(x, parts, b3, W1, b1, W2, b2, Wfc, bfc):
    return pl.pallas_call(
        _mlp_pool_body,
        grid=(NBLK,),
        in_specs=[
            pl.BlockSpec((BN, NF), lambda i: (i, 0)),
            pl.BlockSpec((NC, BN, NF), lambda i: (0, i, 0)),
            pl.BlockSpec((1, 1, BN), lambda i: (i, 0, 0)),
            pl.BlockSpec((NF, H), lambda i: (0, 0)),
            pl.BlockSpec((1, H), lambda i: (0, 0)),
            pl.BlockSpec((H, H), lambda i: (0, 0)),
            pl.BlockSpec((1, H), lambda i: (0, 0)),
            pl.BlockSpec((H, 1), lambda i: (0, 0)),
            pl.BlockSpec((1, 1), lambda i: (0, 0)),
        ],
        out_specs=pl.BlockSpec((GHI, GLO), lambda i: (0, 0)),
        out_shape=jax.ShapeDtypeStruct((GHI, GLO), jnp.float32),
    )(x, parts, b3, W1, b1, W2, b2, Wfc, bfc)


def kernel(pos, z_indices, edge_index, batch, emb, W1, b1, W2, b2, Wfc, bfc):
    z3 = z_indices.astype(jnp.int32).reshape(NBLK, 1, BN)
    b3 = batch.astype(jnp.int32).reshape(NBLK, 1, BN)
    zeros = jnp.zeros((NPAD, NF), jnp.float32)

    x = _build_x(pos, z3, emb)
    parts = _edge_agg(x, edge_index.astype(jnp.int32), zeros)
    pooled = _mlp_pool(x, parts, b3, W1.astype(jnp.float32),
                       b1.reshape(1, H), W2, b2.reshape(1, H),
                       Wfc, bfc.reshape(1, 1))
    return pooled.reshape(NG, 1)


# speedup vs baseline: 1.1796x; 1.1796x over previous
import functools

import jax
import jax.numpy as jnp
from jax import lax
from jax.experimental import pallas as pl
from jax.experimental.pallas import tpu as pltpu
from jax.experimental.pallas import tpu_sc as plsc

N = 100000
E = 6400000
NF = 8
H = 64
VOCAB = 5
NG = 1024
GHI = 32
GLO = 32

N2 = 102400
BN = 2048
NBLK = N2 // BN

NC = 2
NS = 16
NW = NC * NS
SEG = 128
K = 8
CHUNK = K * SEG
NCHUNK = E // CHUNK
NSEGTOT = E // SEG
RPT = N2 // NS
PCH = 1280
NPASS = RPT // PCH
PT128 = PCH // 128
NT128 = N2 // 128


def _sc_body(pos_hbm, z_hbm, emb_hbm, ei_hbm,
             pt_hbm,
             idx_v, rows_v, posb, zb, xb, ab, tb, embb, agg_sh, x_hbm,
             sem_i, sem_g, sem_s):
    c = lax.axis_index("c")
    s = lax.axis_index("s")
    wid = s * NC + c
    r0 = s * RPT
    lane16 = lax.broadcasted_iota(jnp.int32, (16,), 0)

    pltpu.sync_copy(emb_hbm, embb)
    zv16 = jnp.zeros((16,), jnp.float32)

    def zgrp(g, carry):
        rows = g * 16 + lane16
        for f in range(NF):
            plsc.store_scatter(ab, [rows, jnp.full((16,), f, jnp.int32)],
                               zv16)
        return carry

    lax.fori_loop(0, PCH // 16, zgrp, 0)

    @pl.when(c == 1)
    def _():
        for p in range(NPASS):
            pltpu.sync_copy(ab, agg_sh.at[pl.ds(r0 + p * PCH, PCH)])

    for p in range(NPASS):
        nb = r0 + p * PCH
        pltpu.sync_copy(pos_hbm.at[pl.ds(nb * 3, PCH * 3)], posb)
        pltpu.sync_copy(z_hbm.at[pl.ds(nb, PCH)], zb)

        def grp(g, carry):
            base = g * 16
            rows = base + lane16
            zv = zb[pl.ds(base, 16)]
            for f in range(NF):
                if f < 3:
                    v = plsc.load_gather(posb, [rows * 3 + f])
                else:
                    v = plsc.load_gather(
                        embb, [zv, jnp.full((16,), f, jnp.int32)])
                plsc.store_scatter(xb, [rows, jnp.full((16,), f, jnp.int32)],
                                   v)
            return carry

        lax.fori_loop(0, PCH // 16, grp, 0)
        pltpu.sync_copy(xb, x_hbm.at[pl.ds(nb, PCH)])

        @pl.when(c == 0)
        def _():
            pltpu.sync_copy(xb, agg_sh.at[pl.ds(nb, PCH)])

    plsc.subcore_barrier()

    base_n = NCHUNK // NW
    extra = NCHUNK - base_n * NW
    nloc = base_n + jnp.where(wid < extra, 1, 0)

    def idx_copy(i, slot):
        return pltpu.make_async_copy(
            ei_hbm.at[pl.ds((i * NW + wid) * K, K)], idx_v.at[slot], sem_i)

    def scat_copy(j, slot, b):
        return pltpu.make_async_copy(
            rows_v.at[b, j], agg_sh.at[idx_v.at[slot, j, 1]], sem_s)

    idx_copy(0, 0).start()

    def body(i, carry):
        b = jnp.bitwise_and(i, 1)
        s3 = lax.rem(i, 3)

        @pl.when(i + 1 < nloc)
        def _():
            idx_copy(i + 1, lax.rem(i + 1, 3)).start()

        idx_copy(i, s3).wait()
        gs = [pltpu.async_copy(x_hbm.at[idx_v.at[s3, j, 0]],
                               rows_v.at[b, j], sem_g) for j in range(K)]

        @pl.when(i > 0)
        def _():
            pb = jnp.bitwise_xor(b, 1)
            p3 = lax.rem(i + 2, 3)
            for j in range(K):
                scat_copy(j, p3, pb).wait()

        for cp in gs:
            cp.wait()
        for j in range(K):
            scat_copy(j, s3, b).start(add=True)
        return carry

    lax.fori_loop(0, nloc, body, 0)
    lb = jnp.bitwise_and(nloc - 1, 1)
    l3 = lax.rem(nloc - 1, 3)
    for j in range(K):
        scat_copy(j, l3, lb).wait()

    plsc.subcore_barrier()

    for p in range(NPASS):
        nb = r0 + p * PCH
        pltpu.sync_copy(agg_sh.at[pl.ds(nb, PCH)], ab)

        def tgrp(g, carry):
            base = g * 16
            rows = base + lane16
            for f in range(NF):
                v = plsc.load_gather(ab, [rows, jnp.full((16,), f, jnp.int32)])
                tb[f, pl.ds(base, 16)] = v
            return carry

        lax.fori_loop(0, PCH // 16, tgrp, 0)
        for g in range(PT128):
            pltpu.sync_copy(tb.at[:, pl.ds(g * 128, 128)],
                            pt_hbm.at[c, nb // 128 + g])


def _sc_run(pos_flat, z2, embp, ei3):
    mesh = plsc.VectorSubcoreMesh(core_axis_name="c", subcore_axis_name="s")
    fn = functools.partial(
        pl.kernel,
        out_type=jax.ShapeDtypeStruct((NC, NT128, NF, 128), jnp.float32),
        mesh=mesh,
        scratch_types=[
            pltpu.VMEM((3, K, 2, SEG), jnp.int32),
            pltpu.VMEM((2, K, SEG, NF), jnp.float32),
            pltpu.VMEM((PCH * 3,), jnp.float32),
            pltpu.VMEM((PCH,), jnp.int32),
            pltpu.VMEM((PCH, NF), jnp.float32),
            pltpu.VMEM((PCH, NF), jnp.float32),
            pltpu.VMEM((NF, PCH), jnp.float32),
            pltpu.VMEM((VOCAB, NF), jnp.float32),
            pltpu.VMEM_SHARED((N2, NF), jnp.float32),
            pltpu.HBM((N2, NF), jnp.float32),
            pltpu.SemaphoreType.DMA,
            pltpu.SemaphoreType.DMA,
            pltpu.SemaphoreType.DMA,
        ],
        compiler_params=pltpu.CompilerParams(use_tc_tiling_on_sc=False,
                                             needs_layout_passes=False),
    )(_sc_body)
    return fn(pos_flat, z2, embp, ei3)


def _mlp_pool_body(p_ref, b3_ref, W1t_ref, b1_ref, W2t_ref, b2_ref,
                   Wfct_ref, bfc_ref, out_ref):
    i = pl.program_id(0)
    h_t = jnp.concatenate(
        [p_ref[0, t] + p_ref[1, t] for t in range(BN // 128)],
        axis=1)
    a1 = jnp.maximum(
        jnp.dot(W1t_ref[...], h_t, preferred_element_type=jnp.float32)
        + b1_ref[...], 0.0)
    a2 = jnp.maximum(
        jnp.dot(W2t_ref[...], a1, preferred_element_type=jnp.float32)
        + b2_ref[...], 0.0)
    f_t = jnp.dot(Wfct_ref[...], a2,
                  preferred_element_type=jnp.float32)

    seg = b3_ref[0, 0, :]
    lo = jnp.bitwise_and(seg, GLO - 1)
    hi = jnp.right_shift(seg, 5)
    oh_lo = (lo[:, None] == lax.broadcasted_iota(jnp.int32, (BN, GLO), 1))
    oh_hi_t = (hi[None, :] == lax.broadcasted_iota(jnp.int32, (GHI, BN), 0))
    contrib = jnp.dot(oh_hi_t.astype(jnp.float32) * f_t,
                      oh_lo.astype(jnp.float32),
                      preferred_element_type=jnp.float32)

    @pl.when(i == 0)
    def _():
        out_ref[...] = contrib + bfc_ref[...]

    @pl.when(i != 0)
    def _():
        out_ref[...] += contrib


def _mlp_pool(parts_t, b3, W1t, b1c, W2t, b2c, Wfct, bfc):
    return pl.pallas_call(
        _mlp_pool_body,
        grid=(NBLK,),
        in_specs=[
            pl.BlockSpec((NC, BN // 128, NF, 128), lambda i: (0, i, 0, 0)),
            pl.BlockSpec((1, 1, BN), lambda i: (i, 0, 0)),
            pl.BlockSpec((H, NF), lambda i: (0, 0)),
            pl.BlockSpec((H, 1), lambda i: (0, 0)),
            pl.BlockSpec((H, H), lambda i: (0, 0)),
            pl.BlockSpec((H, 1), lambda i: (0, 0)),
            pl.BlockSpec((1, H), lambda i: (0, 0)),
            pl.BlockSpec((1, 1), lambda i: (0, 0)),
        ],
        out_specs=pl.BlockSpec((GHI, GLO), lambda i: (0, 0)),
        out_shape=jax.ShapeDtypeStruct((GHI, GLO), jnp.float32),
    )(parts_t, b3, W1t, b1c, W2t, b2c, Wfct, bfc)


def kernel(pos, z_indices, edge_index, batch, emb, W1, b1, W2, b2, Wfc, bfc):
    pos_flat = jnp.pad(pos, ((0, N2 - N), (0, 0))).reshape(N2 * 3)
    z2 = jnp.pad(z_indices.astype(jnp.int32), (0, N2 - N))
    bat2 = jnp.pad(batch.astype(jnp.int32), (0, N2 - N),
                   constant_values=NG)
    b3 = bat2.reshape(NBLK, 1, BN)
    embp = jnp.pad(emb, ((0, 0), (3, 0)))
    ei3 = jnp.transpose(
        edge_index.astype(jnp.int32).reshape(2, NSEGTOT, SEG), (1, 0, 2))

    parts_t = _sc_run(pos_flat, z2, embp, ei3)
    pooled = _mlp_pool(parts_t, b3, W1.T, b1.reshape(H, 1),
                       W2.T, b2.reshape(H, 1), Wfc.T, bfc.reshape(1, 1))
    return pooled.reshape(NG, 1)


# speedup vs baseline: 1.3889x; 1.1775x over previous
import functools

import jax
import jax.numpy as jnp
from jax import lax
from jax.experimental import pallas as pl
from jax.experimental.pallas import tpu as pltpu
from jax.experimental.pallas import tpu_sc as plsc

N = 100000
E = 6400000
NF = 8
H = 64
VOCAB = 5
NG = 1024
GHI = 32
GLO = 32

N2 = 102400
BN = 2048
NBLK = N2 // BN

NC = 2
NS = 16
NW = NC * NS
SEG = 128
K = 8
CHUNK = K * SEG
NCHUNK = E // CHUNK
NSEGTOT = E // SEG
RPT = N2 // NS
PCH = 1280
NPASS = RPT // PCH
PT128 = PCH // 128
NT128 = N2 // 128


def _sc_body(pos_hbm, z_hbm, emb_hbm, ei_hbm,
             pt_hbm,
             idx_v, rows_v, posb, zb, xb, ab, tb, embb, agg_sh, x_hbm,
             sem_i, sem_g, sem_s):
    c = lax.axis_index("c")
    s = lax.axis_index("s")
    wid = s * NC + c
    r0 = s * RPT
    lane16 = lax.broadcasted_iota(jnp.int32, (16,), 0)

    pltpu.sync_copy(emb_hbm, embb)
    zv16 = jnp.zeros((16,), jnp.float32)

    def zgrp(g, carry):
        rows = g * 16 + lane16
        for f in range(NF):
            plsc.store_scatter(ab, [rows, jnp.full((16,), f, jnp.int32)],
                               zv16)
        return carry

    lax.fori_loop(0, PCH // 16, zgrp, 0)

    @pl.when(c == 1)
    def _():
        for p in range(NPASS):
            pltpu.sync_copy(ab, agg_sh.at[pl.ds(r0 + p * PCH, PCH)])

    for p in range(NPASS):
        nb = r0 + p * PCH
        pltpu.sync_copy(pos_hbm.at[pl.ds(nb // 128, PCH // 128)], posb)
        pltpu.sync_copy(z_hbm.at[pl.ds(nb, PCH)], zb)

        def grp(g, carry):
            base = g * 16
            rows = base + lane16
            q = jnp.right_shift(rows, 7)
            lane = jnp.bitwise_and(rows, 127)
            zv = zb[pl.ds(base, 16)]
            for f in range(NF):
                if f < 3:
                    v = plsc.load_gather(
                        posb, [q, jnp.full((16,), f, jnp.int32), lane])
                else:
                    v = plsc.load_gather(
                        embb, [zv, jnp.full((16,), f, jnp.int32)])
                plsc.store_scatter(xb, [rows, jnp.full((16,), f, jnp.int32)],
                                   v)
            return carry

        lax.fori_loop(0, PCH // 16, grp, 0)
        pltpu.sync_copy(xb, x_hbm.at[pl.ds(nb, PCH)])

        @pl.when(c == 0)
        def _():
            pltpu.sync_copy(xb, agg_sh.at[pl.ds(nb, PCH)])

    plsc.subcore_barrier()

    base_n = NCHUNK // NW
    extra = NCHUNK - base_n * NW
    nloc = base_n + jnp.where(wid < extra, 1, 0)

    def idx_copy(i, slot):
        return pltpu.make_async_copy(
            ei_hbm.at[pl.ds((i * NW + wid) * K, K)], idx_v.at[slot], sem_i)

    def scat_copy(j, slot, b):
        return pltpu.make_async_copy(
            rows_v.at[b, j], agg_sh.at[idx_v.at[slot, j, 1]], sem_s)

    idx_copy(0, 0).start()

    def body(i, carry):
        b = jnp.bitwise_and(i, 1)
        s3 = lax.rem(i, 3)

        @pl.when(i + 1 < nloc)
        def _():
            idx_copy(i + 1, lax.rem(i + 1, 3)).start()

        idx_copy(i, s3).wait()
        gs = [pltpu.async_copy(x_hbm.at[idx_v.at[s3, j, 0]],
                               rows_v.at[b, j], sem_g) for j in range(K)]

        @pl.when(i > 0)
        def _():
            pb = jnp.bitwise_xor(b, 1)
            p3 = lax.rem(i + 2, 3)
            for j in range(K):
                scat_copy(j, p3, pb).wait()

        for cp in gs:
            cp.wait()
        for j in range(K):
            scat_copy(j, s3, b).start(add=True)
        return carry

    lax.fori_loop(0, nloc, body, 0)
    lb = jnp.bitwise_and(nloc - 1, 1)
    l3 = lax.rem(nloc - 1, 3)
    for j in range(K):
        scat_copy(j, l3, lb).wait()

    plsc.subcore_barrier()

    for p in range(NPASS):
        nb = r0 + p * PCH
        pltpu.sync_copy(agg_sh.at[pl.ds(nb, PCH)], ab)

        def tgrp(g, carry):
            base = g * 16
            rows = base + lane16
            for f in range(NF):
                v = plsc.load_gather(ab, [rows, jnp.full((16,), f, jnp.int32)])
                tb[f, pl.ds(base, 16)] = v
            return carry

        lax.fori_loop(0, PCH // 16, tgrp, 0)
        for g in range(PT128):
            pltpu.sync_copy(tb.at[:, pl.ds(g * 128, 128)],
                            pt_hbm.at[c, nb // 128 + g])


def _sc_run(pos_flat, z2, embp, ei3):
    mesh = plsc.VectorSubcoreMesh(core_axis_name="c", subcore_axis_name="s")
    fn = functools.partial(
        pl.kernel,
        out_type=jax.ShapeDtypeStruct((NC, NT128, NF, 128), jnp.float32),
        mesh=mesh,
        scratch_types=[
            pltpu.VMEM((3, K, 2, SEG), jnp.int32),
            pltpu.VMEM((2, K, SEG, NF), jnp.float32),
            pltpu.VMEM((PCH // 128, 4, 128), jnp.float32),
            pltpu.VMEM((PCH,), jnp.int32),
            pltpu.VMEM((PCH, NF), jnp.float32),
            pltpu.VMEM((PCH, NF), jnp.float32),
            pltpu.VMEM((NF, PCH), jnp.float32),
            pltpu.VMEM((VOCAB, NF), jnp.float32),
            pltpu.VMEM_SHARED((N2, NF), jnp.float32),
            pltpu.HBM((N2, NF), jnp.float32),
            pltpu.SemaphoreType.DMA,
            pltpu.SemaphoreType.DMA,
            pltpu.SemaphoreType.DMA,
        ],
        compiler_params=pltpu.CompilerParams(use_tc_tiling_on_sc=False,
                                             needs_layout_passes=False),
    )(_sc_body)
    return fn(pos_flat, z2, embp, ei3)


def _mlp_pool_body(p_ref, b3_ref, W1t_ref, b1_ref, W2t_ref, b2_ref,
                   Wfct_ref, bfc_ref, out_ref):
    i = pl.program_id(0)
    h_t = jnp.concatenate(
        [p_ref[0, t] + p_ref[1, t] for t in range(BN // 128)],
        axis=1)
    a1 = jnp.maximum(
        jnp.dot(W1t_ref[...], h_t, preferred_element_type=jnp.float32)
        + b1_ref[...], 0.0)
    a2 = jnp.maximum(
        jnp.dot(W2t_ref[...], a1, preferred_element_type=jnp.float32)
        + b2_ref[...], 0.0)
    f_t = jnp.dot(Wfct_ref[...], a2,
                  preferred_element_type=jnp.float32)

    seg = b3_ref[0, 0, :]
    lo = jnp.bitwise_and(seg, GLO - 1)
    hi = jnp.right_shift(seg, 5)
    oh_lo = (lo[:, None] == lax.broadcasted_iota(jnp.int32, (BN, GLO), 1))
    oh_hi_t = (hi[None, :] == lax.broadcasted_iota(jnp.int32, (GHI, BN), 0))
    contrib = jnp.dot(oh_hi_t.astype(jnp.float32) * f_t,
                      oh_lo.astype(jnp.float32),
                      preferred_element_type=jnp.float32)

    @pl.when(i == 0)
    def _():
        out_ref[...] = contrib + bfc_ref[...]

    @pl.when(i != 0)
    def _():
        out_ref[...] += contrib


def _mlp_pool(parts_t, b3, W1t, b1c, W2t, b2c, Wfct, bfc):
    return pl.pallas_call(
        _mlp_pool_body,
        grid=(NBLK,),
        in_specs=[
            pl.BlockSpec((NC, BN // 128, NF, 128), lambda i: (0, i, 0, 0)),
            pl.BlockSpec((1, 1, BN), lambda i: (i, 0, 0)),
            pl.BlockSpec((H, NF), lambda i: (0, 0)),
            pl.BlockSpec((H, 1), lambda i: (0, 0)),
            pl.BlockSpec((H, H), lambda i: (0, 0)),
            pl.BlockSpec((H, 1), lambda i: (0, 0)),
            pl.BlockSpec((1, H), lambda i: (0, 0)),
            pl.BlockSpec((1, 1), lambda i: (0, 0)),
        ],
        out_specs=pl.BlockSpec((GHI, GLO), lambda i: (0, 0)),
        out_shape=jax.ShapeDtypeStruct((GHI, GLO), jnp.float32),
    )(parts_t, b3, W1t, b1c, W2t, b2c, Wfct, bfc)


def kernel(pos, z_indices, edge_index, batch, emb, W1, b1, W2, b2, Wfc, bfc):
    pos_tiles = (jnp.pad(pos, ((0, N2 - N), (0, 1))).T
                 .reshape(4, N2 // 128, 128).transpose(1, 0, 2))
    z2 = jnp.pad(z_indices.astype(jnp.int32), (0, N2 - N))
    bat2 = jnp.pad(batch.astype(jnp.int32), (0, N2 - N),
                   constant_values=NG)
    b3 = bat2.reshape(NBLK, 1, BN)
    embp = jnp.pad(emb, ((0, 0), (3, 0)))
    ei3 = jnp.transpose(
        edge_index.astype(jnp.int32).reshape(2, NSEGTOT, SEG), (1, 0, 2))

    parts_t = _sc_run(pos_tiles, z2, embp, ei3)
    pooled = _mlp_pool(parts_t, b3, W1.T, b1.reshape(H, 1),
                       W2.T, b2.reshape(H, 1), Wfc.T, bfc.reshape(1, 1))
    return pooled.reshape(NG, 1)


# speedup vs baseline: 1.4722x; 1.0600x over previous
import functools

import jax
import jax.numpy as jnp
from jax import lax
from jax.experimental import pallas as pl
from jax.experimental.pallas import tpu as pltpu
from jax.experimental.pallas import tpu_sc as plsc

N = 100000
E = 6400000
NF = 8
H = 64
VOCAB = 5
NG = 1024
GHI = 32
GLO = 32

N2 = 102400
BN = 2048
NBLK = N2 // BN

NC = 2
NS = 16
NW = NC * NS
SEG = 128
K = 10
CHUNK = K * SEG
NCHUNK = E // CHUNK
NSEGTOT = E // SEG
RPT = N2 // NS
PCH = 1280
NPASS = RPT // PCH
PT128 = PCH // 128
NT128 = N2 // 128


def _sc_body(pos_hbm, z_hbm, emb_hbm, ei_hbm,
             pt_hbm,
             idx_v, rows_v, posb, zb, xb, ab, tb, embb, agg_sh, x_hbm,
             sem_i, sem_g, sem_s):
    c = lax.axis_index("c")
    s = lax.axis_index("s")
    wid = s * NC + c
    r0 = s * RPT
    lane16 = lax.broadcasted_iota(jnp.int32, (16,), 0)

    pltpu.sync_copy(emb_hbm, embb)
    zv16 = jnp.zeros((16,), jnp.float32)

    def zgrp(g, carry):
        rows = g * 16 + lane16
        for f in range(NF):
            plsc.store_scatter(ab, [rows, jnp.full((16,), f, jnp.int32)],
                               zv16)
        return carry

    lax.fori_loop(0, PCH // 16, zgrp, 0)

    @pl.when(c == 1)
    def _():
        for p in range(NPASS):
            pltpu.sync_copy(ab, agg_sh.at[pl.ds(r0 + p * PCH, PCH)])

    for p in range(NPASS):
        nb = r0 + p * PCH
        pltpu.sync_copy(pos_hbm.at[pl.ds(nb // 128, PCH // 128)], posb)
        pltpu.sync_copy(z_hbm.at[pl.ds(nb, PCH)], zb)

        def grp(g, carry):
            base = g * 16
            rows = base + lane16
            q = jnp.right_shift(rows, 7)
            lane = jnp.bitwise_and(rows, 127)
            zv = zb[pl.ds(base, 16)]
            for f in range(NF):
                if f < 3:
                    v = plsc.load_gather(
                        posb, [q, jnp.full((16,), f, jnp.int32), lane])
                else:
                    v = plsc.load_gather(
                        embb, [zv, jnp.full((16,), f, jnp.int32)])
                plsc.store_scatter(xb, [rows, jnp.full((16,), f, jnp.int32)],
                                   v)
            return carry

        lax.fori_loop(0, PCH // 16, grp, 0)
        pltpu.sync_copy(xb, x_hbm.at[pl.ds(nb, PCH)])

        @pl.when(c == 0)
        def _():
            pltpu.sync_copy(xb, agg_sh.at[pl.ds(nb, PCH)])

    plsc.subcore_barrier()

    base_n = NCHUNK // NW
    extra = NCHUNK - base_n * NW
    nloc = base_n + jnp.where(wid < extra, 1, 0)

    def idx_copy(i, slot):
        return pltpu.make_async_copy(
            ei_hbm.at[pl.ds((i * NW + wid) * K, K)], idx_v.at[slot], sem_i)

    def scat_copy(j, slot, b):
        return pltpu.make_async_copy(
            rows_v.at[b, j], agg_sh.at[idx_v.at[slot, j, 1]], sem_s)

    idx_copy(0, 0).start()

    def body(i, carry):
        b = jnp.bitwise_and(i, 1)
        s3 = lax.rem(i, 3)

        @pl.when(i + 1 < nloc)
        def _():
            idx_copy(i + 1, lax.rem(i + 1, 3)).start()

        idx_copy(i, s3).wait()
        gs = [pltpu.async_copy(x_hbm.at[idx_v.at[s3, j, 0]],
                               rows_v.at[b, j], sem_g) for j in range(K)]

        @pl.when(i > 0)
        def _():
            pb = jnp.bitwise_xor(b, 1)
            p3 = lax.rem(i + 2, 3)
            for j in range(K):
                scat_copy(j, p3, pb).wait()

        for cp in gs:
            cp.wait()
        for j in range(K):
            scat_copy(j, s3, b).start(add=True)
        return carry

    lax.fori_loop(0, nloc, body, 0)
    lb = jnp.bitwise_and(nloc - 1, 1)
    l3 = lax.rem(nloc - 1, 3)
    for j in range(K):
        scat_copy(j, l3, lb).wait()

    plsc.subcore_barrier()

    for p in range(NPASS):
        nb = r0 + p * PCH
        pltpu.sync_copy(agg_sh.at[pl.ds(nb, PCH)], ab)

        def tgrp(g, carry):
            base = g * 16
            rows = base + lane16
            for f in range(NF):
                v = plsc.load_gather(ab, [rows, jnp.full((16,), f, jnp.int32)])
                tb[f, pl.ds(base, 16)] = v
            return carry

        lax.fori_loop(0, PCH // 16, tgrp, 0)
        for g in range(PT128):
            pltpu.sync_copy(tb.at[:, pl.ds(g * 128, 128)],
                            pt_hbm.at[c, nb // 128 + g])


def _sc_run(pos_flat, z2, embp, ei3):
    mesh = plsc.VectorSubcoreMesh(core_axis_name="c", subcore_axis_name="s")
    fn = functools.partial(
        pl.kernel,
        out_type=jax.ShapeDtypeStruct((NC, NT128, NF, 128), jnp.float32),
        mesh=mesh,
        scratch_types=[
            pltpu.VMEM((3, K, 2, SEG), jnp.int32),
            pltpu.VMEM((2, K, SEG, NF), jnp.float32),
            pltpu.VMEM((PCH // 128, 4, 128), jnp.float32),
            pltpu.VMEM((PCH,), jnp.int32),
            pltpu.VMEM((PCH, NF), jnp.float32),
            pltpu.VMEM((PCH, NF), jnp.float32),
            pltpu.VMEM((NF, PCH), jnp.float32),
            pltpu.VMEM((VOCAB, NF), jnp.float32),
            pltpu.VMEM_SHARED((N2, NF), jnp.float32),
            pltpu.HBM((N2, NF), jnp.float32),
            pltpu.SemaphoreType.DMA,
            pltpu.SemaphoreType.DMA,
            pltpu.SemaphoreType.DMA,
        ],
        compiler_params=pltpu.CompilerParams(use_tc_tiling_on_sc=False,
                                             needs_layout_passes=False),
    )(_sc_body)
    return fn(pos_flat, z2, embp, ei3)


def _mlp_pool_body(p_ref, b3_ref, W1t_ref, b1_ref, W2t_ref, b2_ref,
                   Wfct_ref, bfc_ref, out_ref):
    i = pl.program_id(0)
    h_t = jnp.concatenate(
        [p_ref[0, t] + p_ref[1, t] for t in range(BN // 128)],
        axis=1)
    a1 = jnp.maximum(
        jnp.dot(W1t_ref[...], h_t, preferred_element_type=jnp.float32)
        + b1_ref[...], 0.0)
    a2 = jnp.maximum(
        jnp.dot(W2t_ref[...], a1, preferred_element_type=jnp.float32)
        + b2_ref[...], 0.0)
    f_t = jnp.dot(Wfct_ref[...], a2,
                  preferred_element_type=jnp.float32)

    seg = b3_ref[0, 0, :]
    lo = jnp.bitwise_and(seg, GLO - 1)
    hi = jnp.right_shift(seg, 5)
    oh_lo = (lo[:, None] == lax.broadcasted_iota(jnp.int32, (BN, GLO), 1))
    oh_hi_t = (hi[None, :] == lax.broadcasted_iota(jnp.int32, (GHI, BN), 0))
    contrib = jnp.dot(oh_hi_t.astype(jnp.float32) * f_t,
                      oh_lo.astype(jnp.float32),
                      preferred_element_type=jnp.float32)

    @pl.when(i == 0)
    def _():
        out_ref[...] = contrib + bfc_ref[...]

    @pl.when(i != 0)
    def _():
        out_ref[...] += contrib


def _mlp_pool(parts_t, b3, W1t, b1c, W2t, b2c, Wfct, bfc):
    return pl.pallas_call(
        _mlp_pool_body,
        grid=(NBLK,),
        in_specs=[
            pl.BlockSpec((NC, BN // 128, NF, 128), lambda i: (0, i, 0, 0)),
            pl.BlockSpec((1, 1, BN), lambda i: (i, 0, 0)),
            pl.BlockSpec((H, NF), lambda i: (0, 0)),
            pl.BlockSpec((H, 1), lambda i: (0, 0)),
            pl.BlockSpec((H, H), lambda i: (0, 0)),
            pl.BlockSpec((H, 1), lambda i: (0, 0)),
            pl.BlockSpec((1, H), lambda i: (0, 0)),
            pl.BlockSpec((1, 1), lambda i: (0, 0)),
        ],
        out_specs=pl.BlockSpec((GHI, GLO), lambda i: (0, 0)),
        out_shape=jax.ShapeDtypeStruct((GHI, GLO), jnp.float32),
    )(parts_t, b3, W1t, b1c, W2t, b2c, Wfct, bfc)


def kernel(pos, z_indices, edge_index, batch, emb, W1, b1, W2, b2, Wfc, bfc):
    pos_tiles = (jnp.pad(pos, ((0, N2 - N), (0, 1))).T
                 .reshape(4, N2 // 128, 128).transpose(1, 0, 2))
    z2 = jnp.pad(z_indices.astype(jnp.int32), (0, N2 - N))
    bat2 = jnp.pad(batch.astype(jnp.int32), (0, N2 - N),
                   constant_values=NG)
    b3 = bat2.reshape(NBLK, 1, BN)
    embp = jnp.pad(emb, ((0, 0), (3, 0)))
    ei3 = jnp.transpose(
        edge_index.astype(jnp.int32).reshape(2, NSEGTOT, SEG), (1, 0, 2))

    parts_t = _sc_run(pos_tiles, z2, embp, ei3)
    pooled = _mlp_pool(parts_t, b3, W1.T, b1.reshape(H, 1),
                       W2.T, b2.reshape(H, 1), Wfc.T, bfc.reshape(1, 1))
    return pooled.reshape(NG, 1)
